# baseline (speedup 1.0000x reference)
import math

import jax
import jax.numpy as jnp
from jax.experimental import pallas as pl
from jax.experimental.pallas import tpu as pltpu

_LAMDA = 1.5
_ALPHA = 0.7
_PAIR = 2


def _bdot(a, b):
    return jnp.dot(
        a.astype(jnp.bfloat16),
        b.astype(jnp.bfloat16),
        preferred_element_type=jnp.float32,
    )


def _fused_body(x_ref, adj_ref, w0_ref, b0_ref, w1_ref, w2_ref, w3_ref, o_ref):
    adj = adj_ref[...].astype(jnp.bfloat16)
    b0 = b0_ref[...]
    hs = []
    for p in range(_PAIR):
        h = _bdot(x_ref[p], w0_ref[...])
        hs.append(jnp.maximum(h + b0, 0.0))
    h0s = list(hs)
    for i, w_ref in enumerate((w1_ref, w2_ref, w3_ref), start=1):
        theta = min(1.0, math.log(_LAMDA / i + 1.0))
        w = w_ref[...]
        for p in range(_PAIR):
            hi = jnp.dot(adj, hs[p].astype(jnp.bfloat16), preferred_element_type=jnp.float32)
            support = (1.0 - _ALPHA) * hi + _ALPHA * h0s[p]
            out = theta * _bdot(support, w)
            out = out + (1.0 - theta) * support + hs[p]
            hs[p] = jnp.maximum(out, 0.0)
    for p in range(_PAIR):
        o_ref[p] = hs[p]


def kernel(x, adj, W0, b0, W1, W2, W3):
    B, N, F = x.shape
    H = W0.shape[1]
    b0_2d = b0.reshape(1, H)

    return pl.pallas_call(
        _fused_body,
        grid=(B // _PAIR,),
        in_specs=[
            pl.BlockSpec((_PAIR, N, F), lambda b: (b, 0, 0)),
            pl.BlockSpec((N, N), lambda b: (0, 0)),
            pl.BlockSpec((F, H), lambda b: (0, 0)),
            pl.BlockSpec((1, H), lambda b: (0, 0)),
            pl.BlockSpec((H, H), lambda b: (0, 0)),
            pl.BlockSpec((H, H), lambda b: (0, 0)),
            pl.BlockSpec((H, H), lambda b: (0, 0)),
        ],
        out_specs=pl.BlockSpec((_PAIR, N, H), lambda b: (b, 0, 0)),
        out_shape=jax.ShapeDtypeStruct((B, N, H), jnp.float32),
        compiler_params=pltpu.CompilerParams(
            dimension_semantics=("parallel",),
        ),
    )(x, adj, W0, b0_2d, W1, W2, W3)


# speedup vs baseline: 1.0338x; 1.0338x over previous
import math

import jax
import jax.numpy as jnp
from jax.experimental import pallas as pl
from jax.experimental.pallas import tpu as pltpu

_LAMDA = 1.5
_ALPHA = 0.7


def _bdot(a, b):
    return jnp.dot(
        a.astype(jnp.bfloat16),
        b.astype(jnp.bfloat16),
        preferred_element_type=jnp.float32,
    )


def _fused_body(x_ref, adj_ref, w0_ref, b0_ref, w1_ref, w2_ref, w3_ref, o_ref):
    x = x_ref[0]
    h = _bdot(x, w0_ref[...])
    h = jnp.maximum(h + b0_ref[...], 0.0)
    h0 = h
    adj = adj_ref[...].astype(jnp.bfloat16)
    for i, w_ref in enumerate((w1_ref, w2_ref, w3_ref), start=1):
        theta = min(1.0, math.log(_LAMDA / i + 1.0))
        hi = jnp.dot(adj, h.astype(jnp.bfloat16), preferred_element_type=jnp.float32)
        support = (1.0 - _ALPHA) * hi + _ALPHA * h0
        out = theta * _bdot(support, w_ref[...])
        out = out + (1.0 - theta) * support + h
        h = jnp.maximum(out, 0.0)
    o_ref[0] = h


def kernel(x, adj, W0, b0, W1, W2, W3):
    B, N, F = x.shape
    H = W0.shape[1]
    b0_2d = b0.reshape(1, H)

    return pl.pallas_call(
        _fused_body,
        grid=(B,),
        in_specs=[
            pl.BlockSpec((1, N, F), lambda b: (b, 0, 0)),
            pl.BlockSpec((N, N), lambda b: (0, 0)),
            pl.BlockSpec((F, H), lambda b: (0, 0)),
            pl.BlockSpec((1, H), lambda b: (0, 0)),
            pl.BlockSpec((H, H), lambda b: (0, 0)),
            pl.BlockSpec((H, H), lambda b: (0, 0)),
            pl.BlockSpec((H, H), lambda b: (0, 0)),
        ],
        out_specs=pl.BlockSpec((1, N, H), lambda b: (b, 0, 0)),
        out_shape=jax.ShapeDtypeStruct((B, N, H), jnp.float32),
        compiler_params=pltpu.CompilerParams(
            dimension_semantics=("parallel",),
        ),
    )(x, adj, W0, b0_2d, W1, W2, W3)
